# baseline (speedup 1.0000x reference)
import functools

import jax
import jax.numpy as jnp
from jax.experimental import pallas as pl
from jax.experimental.pallas import tpu as pltpu
from jax.experimental.pallas import tpu_sc as plsc

VOCAB, EMBED, HIDDEN = 100000, 256, 512
B, T = 16, 20
BT = B * T
NPAD = 512
GWIN = 128
VTILE = 4096


def _gather_call(E, idx):
    mesh = plsc.VectorSubcoreMesh(core_axis_name="c", subcore_axis_name="s")

    @functools.partial(
        pl.kernel,
        out_type=jax.ShapeDtypeStruct((NPAD, EMBED), E.dtype),
        mesh=mesh,
    )
    def gather_kernel(e_hbm, i_hbm, o_hbm):
        def body(i_vmem, o_vmem):
            pltpu.sync_copy(e_hbm.at[i_vmem.at[0]], o_vmem)

        pltpu.emit_pipeline(
            body,
            grid=(NPAD // GWIN,),
            in_specs=[pl.BlockSpec((1, GWIN), lambda i: (0, i))],
            out_specs=[pl.BlockSpec((GWIN, EMBED), lambda i: (i, 0))],
            core_axis_name="s",
            dimension_semantics=(pltpu.PARALLEL,),
        )(i_hbm, o_hbm)

    return gather_kernel(E, idx)


def _gru_proj_body(emb_ref, enc_ref, wih0_ref, whh0_ref, bi0_ref, bh0_ref,
                   wih1_ref, whh1_ref, bi1_ref, bh1_ref,
                   w1_ref, b1_ref, w2_ref, b2_ref, out_ref, ys_ref):
    def gru(x, h, wih, whh, bi, bh):
        gi = jnp.dot(x, wih, preferred_element_type=jnp.float32) + bi
        gh = jnp.dot(h, whh, preferred_element_type=jnp.float32) + bh
        i_r, i_z, i_n = gi[:, :HIDDEN], gi[:, HIDDEN:2 * HIDDEN], gi[:, 2 * HIDDEN:]
        h_r, h_z, h_n = gh[:, :HIDDEN], gh[:, HIDDEN:2 * HIDDEN], gh[:, 2 * HIDDEN:]
        r = jax.nn.sigmoid(i_r + h_r)
        z = jax.nn.sigmoid(i_z + h_z)
        n = jnp.tanh(i_n + r * h_n)
        return (1.0 - z) * n + z * h

    def step(t, carry):
        h0, h1 = carry
        x = emb_ref[pl.ds(t * B, B), :]
        h0n = gru(x, h0, wih0_ref[...], whh0_ref[...], bi0_ref[...], bh0_ref[...])
        h1n = gru(h0n, h1, wih1_ref[...], whh1_ref[...], bi1_ref[...], bh1_ref[...])
        ys_ref[pl.ds(t * B, B), :] = h1n
        return (h0n, h1n)

    h0 = enc_ref[0]
    h1 = enc_ref[1]
    jax.lax.fori_loop(0, T, step, (h0, h1))
    ys = ys_ref[...]
    hid = jnp.tanh(jnp.dot(ys, w1_ref[...], preferred_element_type=jnp.float32)
                   + b1_ref[...])
    out_ref[...] = jnp.dot(hid, w2_ref[...],
                           preferred_element_type=jnp.float32) + b2_ref[...]


def _logits_body(proj_ref, e_ref, gb_ref, out_ref):
    out_ref[...] = jax.lax.dot_general(
        proj_ref[...], e_ref[...],
        (((1,), (1,)), ((), ())),
        preferred_element_type=jnp.float32) + gb_ref[...]


def kernel(encoding, trg, E, W_ih0, W_hh0, b_ih0, b_hh0, W_ih1, W_hh1,
           b_ih1, b_hh1, W1, b1, W2, b2, g_b):
    idx = trg.astype(jnp.int32).T.reshape(-1)
    idx = jnp.concatenate([idx, jnp.zeros((NPAD - BT,), jnp.int32)])
    idx = idx.reshape(1, NPAD)

    emb = _gather_call(E, idx)

    gru_weights = (
        encoding,
        W_ih0.T, W_hh0.T, b_ih0.reshape(1, -1), b_hh0.reshape(1, -1),
        W_ih1.T, W_hh1.T, b_ih1.reshape(1, -1), b_hh1.reshape(1, -1),
        W1.T, b1.reshape(1, -1), W2.T, b2.reshape(1, -1),
    )

    proj = pl.pallas_call(
        _gru_proj_body,
        out_shape=jax.ShapeDtypeStruct((BT, EMBED), jnp.float32),
        scratch_shapes=[pltpu.VMEM((BT, HIDDEN), jnp.float32)],
    )(emb, *gru_weights)

    proj_bt = proj.reshape(T, B, EMBED).transpose(1, 0, 2).reshape(BT, EMBED)

    nv = pl.cdiv(VOCAB, VTILE)
    logits = pl.pallas_call(
        _logits_body,
        grid=(nv,),
        in_specs=[
            pl.BlockSpec((BT, EMBED), lambda i: (0, 0)),
            pl.BlockSpec((VTILE, EMBED), lambda i: (i, 0)),
            pl.BlockSpec((1, VTILE), lambda i: (0, i)),
        ],
        out_specs=pl.BlockSpec((BT, VTILE), lambda i: (0, i)),
        out_shape=jax.ShapeDtypeStruct((BT, VOCAB), jnp.float32),
        compiler_params=pltpu.CompilerParams(
            dimension_semantics=("arbitrary",)),
    )(proj_bt, E, g_b.reshape(1, VOCAB))

    return logits.reshape(B, T, VOCAB)


# speedup vs baseline: 1.5210x; 1.5210x over previous
import functools

import jax
import jax.numpy as jnp
from jax.experimental import pallas as pl
from jax.experimental.pallas import tpu as pltpu
from jax.experimental.pallas import tpu_sc as plsc

VOCAB, EMBED, HIDDEN = 100000, 256, 512
B, T = 16, 20
BT = B * T
NPAD = 512
GWIN = 128
VTILE = 4096


def _gather_call(E, idx):
    mesh = plsc.VectorSubcoreMesh(core_axis_name="c", subcore_axis_name="s")

    @functools.partial(
        pl.kernel,
        out_type=jax.ShapeDtypeStruct((NPAD, EMBED), E.dtype),
        mesh=mesh,
    )
    def gather_kernel(e_hbm, i_hbm, o_hbm):
        def body(i_vmem, o_vmem):
            pltpu.sync_copy(e_hbm.at[i_vmem.at[0]], o_vmem)

        pltpu.emit_pipeline(
            body,
            grid=(NPAD // GWIN,),
            in_specs=[pl.BlockSpec((1, GWIN), lambda i: (0, i))],
            out_specs=[pl.BlockSpec((GWIN, EMBED), lambda i: (i, 0))],
            core_axis_name="s",
            dimension_semantics=(pltpu.PARALLEL,),
        )(i_hbm, o_hbm)

    return gather_kernel(E, idx)


def _gru_proj_body(emb_ref, enc_ref, wih0_ref, whh0_ref, bi0_ref, bh0_ref,
                   wih1_ref, whh1_ref, bi1_ref, bh1_ref,
                   w1_ref, b1_ref, w2_ref, b2_ref, out_ref, ys_ref):
    def gru(x, h, wih, whh, bi, bh):
        gi = jnp.dot(x, wih, preferred_element_type=jnp.float32) + bi
        gh = jnp.dot(h, whh, preferred_element_type=jnp.float32) + bh
        i_r, i_z, i_n = gi[:, :HIDDEN], gi[:, HIDDEN:2 * HIDDEN], gi[:, 2 * HIDDEN:]
        h_r, h_z, h_n = gh[:, :HIDDEN], gh[:, HIDDEN:2 * HIDDEN], gh[:, 2 * HIDDEN:]
        r = jax.nn.sigmoid(i_r + h_r)
        z = jax.nn.sigmoid(i_z + h_z)
        n = jnp.tanh(i_n + r * h_n)
        return (1.0 - z) * n + z * h

    def step(t, carry):
        h0, h1 = carry
        x = emb_ref[pl.ds(t * B, B), :]
        h0n = gru(x, h0, wih0_ref[...], whh0_ref[...], bi0_ref[...], bh0_ref[...])
        h1n = gru(h0n, h1, wih1_ref[...], whh1_ref[...], bi1_ref[...], bh1_ref[...])
        ys_ref[pl.ds(t * B, B), :] = h1n
        return (h0n, h1n)

    h0 = enc_ref[0]
    h1 = enc_ref[1]
    jax.lax.fori_loop(0, T, step, (h0, h1))
    ys = ys_ref[...]
    hid = jnp.tanh(jnp.dot(ys, w1_ref[...], preferred_element_type=jnp.float32)
                   + b1_ref[...])
    out_ref[...] = jnp.dot(hid, w2_ref[...],
                           preferred_element_type=jnp.float32) + b2_ref[...]


def _logits_body(proj_ref, e_ref, gb_ref, out_ref):
    res = jax.lax.dot_general(
        proj_ref[...], e_ref[...],
        (((1,), (1,)), ((), ())),
        preferred_element_type=jnp.float32) + gb_ref[...]
    out_ref[...] = res.reshape(B, T, res.shape[-1])


def kernel(encoding, trg, E, W_ih0, W_hh0, b_ih0, b_hh0, W_ih1, W_hh1,
           b_ih1, b_hh1, W1, b1, W2, b2, g_b):
    idx = trg.astype(jnp.int32).T.reshape(-1)
    idx = jnp.concatenate([idx, jnp.zeros((NPAD - BT,), jnp.int32)])
    idx = idx.reshape(1, NPAD)

    emb = _gather_call(E, idx)

    gru_weights = (
        encoding,
        W_ih0.T, W_hh0.T, b_ih0.reshape(1, -1), b_hh0.reshape(1, -1),
        W_ih1.T, W_hh1.T, b_ih1.reshape(1, -1), b_hh1.reshape(1, -1),
        W1.T, b1.reshape(1, -1), W2.T, b2.reshape(1, -1),
    )

    proj = pl.pallas_call(
        _gru_proj_body,
        out_shape=jax.ShapeDtypeStruct((BT, EMBED), jnp.float32),
        scratch_shapes=[pltpu.VMEM((BT, HIDDEN), jnp.float32)],
    )(emb, *gru_weights)

    proj_bt = proj.reshape(T, B, EMBED).transpose(1, 0, 2).reshape(BT, EMBED)

    nv = pl.cdiv(VOCAB, VTILE)
    logits = pl.pallas_call(
        _logits_body,
        grid=(nv,),
        in_specs=[
            pl.BlockSpec((BT, EMBED), lambda i: (0, 0)),
            pl.BlockSpec((VTILE, EMBED), lambda i: (i, 0)),
            pl.BlockSpec((1, VTILE), lambda i: (0, i)),
        ],
        out_specs=pl.BlockSpec((B, T, VTILE), lambda i: (0, 0, i)),
        out_shape=jax.ShapeDtypeStruct((B, T, VOCAB), jnp.float32),
        compiler_params=pltpu.CompilerParams(
            dimension_semantics=("arbitrary",)),
    )(proj_bt, E, g_b.reshape(1, VOCAB))

    return logits


# speedup vs baseline: 1.6657x; 1.0951x over previous
import jax
import jax.numpy as jnp
from jax.experimental import pallas as pl
from jax.experimental.pallas import tpu as pltpu

VOCAB, EMBED, HIDDEN = 100000, 256, 512
B, T = 16, 20
BT = B * T
VTILE = 4096
LOOKAHEAD = 2


def _gru_proj_body(idx_ref, e_any, enc_ref, wih0_ref, whh0_ref, bi0_ref,
                   bh0_ref, wih1_ref, whh1_ref, bi1_ref, bh1_ref,
                   w1_ref, b1_ref, w2_ref, b2_ref, out_ref,
                   emb_ref, ys_ref, sem):
    def row_copy(t, b):
        idx = idx_ref[t, b]
        return pltpu.make_async_copy(
            e_any.at[pl.ds(idx, 1), :],
            emb_ref.at[pl.ds(t * B + b, 1), :],
            sem)

    def issue_step(t):
        for b in range(B):
            row_copy(t, b).start()

    def wait_step(t):
        for b in range(B):
            row_copy(t, b).wait()

    for t in range(LOOKAHEAD):
        issue_step(t)

    def gru(x, h, wih, whh, bi, bh):
        gi = jnp.dot(x, wih, preferred_element_type=jnp.float32) + bi
        gh = jnp.dot(h, whh, preferred_element_type=jnp.float32) + bh
        i_r, i_z, i_n = gi[:, :HIDDEN], gi[:, HIDDEN:2 * HIDDEN], gi[:, 2 * HIDDEN:]
        h_r, h_z, h_n = gh[:, :HIDDEN], gh[:, HIDDEN:2 * HIDDEN], gh[:, 2 * HIDDEN:]
        r = jax.nn.sigmoid(i_r + h_r)
        z = jax.nn.sigmoid(i_z + h_z)
        n = jnp.tanh(i_n + r * h_n)
        return (1.0 - z) * n + z * h

    def step(t, carry):
        h0, h1 = carry

        @pl.when(t < T - LOOKAHEAD)
        def _():
            issue_step(t + LOOKAHEAD)

        wait_step(t)
        x = emb_ref[pl.ds(t * B, B), :]
        h0n = gru(x, h0, wih0_ref[...], whh0_ref[...], bi0_ref[...], bh0_ref[...])
        h1n = gru(h0n, h1, wih1_ref[...], whh1_ref[...], bi1_ref[...], bh1_ref[...])
        ys_ref[pl.ds(t * B, B), :] = h1n
        return (h0n, h1n)

    h0 = enc_ref[0]
    h1 = enc_ref[1]
    jax.lax.fori_loop(0, T, step, (h0, h1))
    ys = ys_ref[...]
    hid = jnp.tanh(jnp.dot(ys, w1_ref[...], preferred_element_type=jnp.float32)
                   + b1_ref[...])
    out_ref[...] = jnp.dot(hid, w2_ref[...],
                           preferred_element_type=jnp.float32) + b2_ref[...]


def _logits_body(proj_ref, e_ref, gb_ref, out_ref):
    res = jax.lax.dot_general(
        proj_ref[...], e_ref[...],
        (((1,), (1,)), ((), ())),
        preferred_element_type=jnp.float32) + gb_ref[...]
    out_ref[...] = res.reshape(B, T, res.shape[-1])


def kernel(encoding, trg, E, W_ih0, W_hh0, b_ih0, b_hh0, W_ih1, W_hh1,
           b_ih1, b_hh1, W1, b1, W2, b2, g_b):
    idx = trg.astype(jnp.int32).T

    gru_weights = (
        encoding,
        W_ih0.T, W_hh0.T, b_ih0.reshape(1, -1), b_hh0.reshape(1, -1),
        W_ih1.T, W_hh1.T, b_ih1.reshape(1, -1), b_hh1.reshape(1, -1),
        W1.T, b1.reshape(1, -1), W2.T, b2.reshape(1, -1),
    )

    proj = pl.pallas_call(
        _gru_proj_body,
        in_specs=[pl.BlockSpec(memory_space=pltpu.SMEM),
                  pl.BlockSpec(memory_space=pltpu.MemorySpace.HBM)]
                 + [pl.BlockSpec(memory_space=pltpu.MemorySpace.VMEM)] * 13,
        out_shape=jax.ShapeDtypeStruct((BT, EMBED), jnp.float32),
        scratch_shapes=[pltpu.VMEM((BT, EMBED), jnp.float32),
                        pltpu.VMEM((BT, HIDDEN), jnp.float32),
                        pltpu.SemaphoreType.DMA],
    )(idx, E, *gru_weights)

    proj_bt = proj.reshape(T, B, EMBED).transpose(1, 0, 2).reshape(BT, EMBED)

    nv = pl.cdiv(VOCAB, VTILE)
    logits = pl.pallas_call(
        _logits_body,
        grid=(nv,),
        in_specs=[
            pl.BlockSpec((BT, EMBED), lambda i: (0, 0)),
            pl.BlockSpec((VTILE, EMBED), lambda i: (i, 0)),
            pl.BlockSpec((1, VTILE), lambda i: (0, i)),
        ],
        out_specs=pl.BlockSpec((B, T, VTILE), lambda i: (0, 0, i)),
        out_shape=jax.ShapeDtypeStruct((B, T, VOCAB), jnp.float32),
        compiler_params=pltpu.CompilerParams(
            dimension_semantics=("arbitrary",)),
    )(proj_bt, E, g_b.reshape(1, VOCAB))

    return logits


# speedup vs baseline: 3.0749x; 1.8460x over previous
import jax
import jax.numpy as jnp
from jax.experimental import pallas as pl
from jax.experimental.pallas import tpu as pltpu

VOCAB, EMBED, HIDDEN = 100000, 256, 512
B, T = 16, 20
BT = B * T
VTILE = 4096
LOOKAHEAD = 2

_NT = (((1,), (1,)), ((), ()))


def _gru_proj_body(idx_ref, e_any, enc_ref, wih0_ref, whh0_ref, bi0_ref,
                   bh0_ref, wih1_ref, whh1_ref, bi1_ref, bh1_ref,
                   w1_ref, b1_ref, w2_ref, b2_ref, out_ref,
                   emb_ref, ys_ref, sem):
    def row_copy(t, b):
        idx = idx_ref[b, t]
        return pltpu.make_async_copy(
            e_any.at[pl.ds(idx, 1), :],
            emb_ref.at[pl.ds(t * B + b, 1), :],
            sem)

    def issue_step(t):
        for b in range(B):
            row_copy(t, b).start()

    def wait_step(t):
        for b in range(B):
            row_copy(t, b).wait()

    for t in range(LOOKAHEAD):
        issue_step(t)

    def gru(x, h, wih, whh, bi, bh):
        gi = jax.lax.dot_general(x, wih, _NT,
                                 preferred_element_type=jnp.float32) + bi
        gh = jax.lax.dot_general(h, whh, _NT,
                                 preferred_element_type=jnp.float32) + bh
        i_r, i_z, i_n = gi[:, :HIDDEN], gi[:, HIDDEN:2 * HIDDEN], gi[:, 2 * HIDDEN:]
        h_r, h_z, h_n = gh[:, :HIDDEN], gh[:, HIDDEN:2 * HIDDEN], gh[:, 2 * HIDDEN:]
        r = jax.nn.sigmoid(i_r + h_r)
        z = jax.nn.sigmoid(i_z + h_z)
        n = jnp.tanh(i_n + r * h_n)
        return (1.0 - z) * n + z * h

    def step(t, carry):
        h0, h1 = carry

        @pl.when(t < T - LOOKAHEAD)
        def _():
            issue_step(t + LOOKAHEAD)

        wait_step(t)
        x = emb_ref[pl.ds(t * B, B), :]
        h0n = gru(x, h0, wih0_ref[...], whh0_ref[...], bi0_ref[...], bh0_ref[...])
        h1n = gru(h0n, h1, wih1_ref[...], whh1_ref[...], bi1_ref[...], bh1_ref[...])
        ys_ref[pl.ds(t * B, B), :] = h1n
        return (h0n, h1n)

    h0 = enc_ref[0]
    h1 = enc_ref[1]
    jax.lax.fori_loop(0, T, step, (h0, h1))
    ys = ys_ref[...]
    hid = jnp.tanh(jax.lax.dot_general(ys, w1_ref[...], _NT,
                                       preferred_element_type=jnp.float32)
                   + b1_ref[...])
    out_ref[...] = jax.lax.dot_general(hid, w2_ref[...], _NT,
                                       preferred_element_type=jnp.float32) + b2_ref[...]


def _logits_body(proj_ref, e_ref, gb_ref, out_ref):
    res = jax.lax.dot_general(
        proj_ref[...], e_ref[...], _NT,
        preferred_element_type=jnp.float32) + gb_ref[...]
    out_ref[...] = res.reshape(T, B, res.shape[-1])


def kernel(encoding, trg, E, W_ih0, W_hh0, b_ih0, b_hh0, W_ih1, W_hh1,
           b_ih1, b_hh1, W1, b1, W2, b2, g_b):
    idx = trg.astype(jnp.int32)

    gru_weights = (
        encoding,
        W_ih0, W_hh0, b_ih0.reshape(1, -1), b_hh0.reshape(1, -1),
        W_ih1, W_hh1, b_ih1.reshape(1, -1), b_hh1.reshape(1, -1),
        W1, b1.reshape(1, -1), W2, b2.reshape(1, -1),
    )

    proj = pl.pallas_call(
        _gru_proj_body,
        in_specs=[pl.BlockSpec(memory_space=pltpu.SMEM),
                  pl.BlockSpec(memory_space=pltpu.MemorySpace.HBM)]
                 + [pl.BlockSpec(memory_space=pltpu.MemorySpace.VMEM)] * 13,
        out_shape=jax.ShapeDtypeStruct((BT, EMBED), jnp.float32),
        scratch_shapes=[pltpu.VMEM((BT, EMBED), jnp.float32),
                        pltpu.VMEM((BT, HIDDEN), jnp.float32),
                        pltpu.SemaphoreType.DMA],
    )(idx, E, *gru_weights)

    nv = pl.cdiv(VOCAB, VTILE)
    logits_tb = pl.pallas_call(
        _logits_body,
        grid=(nv,),
        in_specs=[
            pl.BlockSpec((BT, EMBED), lambda i: (0, 0)),
            pl.BlockSpec((VTILE, EMBED), lambda i: (i, 0)),
            pl.BlockSpec((1, VTILE), lambda i: (0, i)),
        ],
        out_specs=pl.BlockSpec((T, B, VTILE), lambda i: (0, 0, i)),
        out_shape=jax.ShapeDtypeStruct((T, B, VOCAB), jnp.float32),
        compiler_params=pltpu.CompilerParams(
            dimension_semantics=("arbitrary",)),
    )(proj, E, g_b.reshape(1, VOCAB))

    return logits_tb.transpose(1, 0, 2)


# speedup vs baseline: 3.1321x; 1.0186x over previous
import jax
import jax.numpy as jnp
from jax.experimental import pallas as pl
from jax.experimental.pallas import tpu as pltpu

VOCAB, EMBED, HIDDEN = 100000, 256, 512
B, T = 16, 20
BT = B * T
VTILE = 8192
LOOKAHEAD = 2

_NT = (((1,), (1,)), ((), ()))


def _gru_proj_body(idx_ref, e_any, enc_ref, wih0_ref, whh0_ref, bi0_ref,
                   bh0_ref, wih1_ref, whh1_ref, bi1_ref, bh1_ref,
                   w1_ref, b1_ref, w2_ref, b2_ref, out_ref,
                   emb_ref, ys_ref, sem):
    def row_copy(t, b):
        idx = idx_ref[b, t]
        return pltpu.make_async_copy(
            e_any.at[pl.ds(idx, 1), :],
            emb_ref.at[pl.ds(t * B + b, 1), :],
            sem)

    def issue_step(t):
        for b in range(B):
            row_copy(t, b).start()

    def wait_step(t):
        for b in range(B):
            row_copy(t, b).wait()

    for t in range(LOOKAHEAD):
        issue_step(t)

    def gru(x, h, wih, whh, bi, bh):
        gi = jax.lax.dot_general(x, wih, _NT,
                                 preferred_element_type=jnp.float32) + bi
        gh = jax.lax.dot_general(h, whh, _NT,
                                 preferred_element_type=jnp.float32) + bh
        i_r, i_z, i_n = gi[:, :HIDDEN], gi[:, HIDDEN:2 * HIDDEN], gi[:, 2 * HIDDEN:]
        h_r, h_z, h_n = gh[:, :HIDDEN], gh[:, HIDDEN:2 * HIDDEN], gh[:, 2 * HIDDEN:]
        r = jax.nn.sigmoid(i_r + h_r)
        z = jax.nn.sigmoid(i_z + h_z)
        n = jnp.tanh(i_n + r * h_n)
        return (1.0 - z) * n + z * h

    def step(t, carry):
        h0, h1 = carry

        @pl.when(t < T - LOOKAHEAD)
        def _():
            issue_step(t + LOOKAHEAD)

        wait_step(t)
        x = emb_ref[pl.ds(t * B, B), :]
        h0n = gru(x, h0, wih0_ref[...], whh0_ref[...], bi0_ref[...], bh0_ref[...])
        h1n = gru(h0n, h1, wih1_ref[...], whh1_ref[...], bi1_ref[...], bh1_ref[...])
        ys_ref[pl.ds(t * B, B), :] = h1n
        return (h0n, h1n)

    h0 = enc_ref[0]
    h1 = enc_ref[1]
    jax.lax.fori_loop(0, T, step, (h0, h1))
    ys = ys_ref[...]
    hid = jnp.tanh(jax.lax.dot_general(ys, w1_ref[...], _NT,
                                       preferred_element_type=jnp.float32)
                   + b1_ref[...])
    out_ref[...] = jax.lax.dot_general(hid, w2_ref[...], _NT,
                                       preferred_element_type=jnp.float32) + b2_ref[...]


def _logits_body(proj_ref, e_ref, gb_ref, out_ref):
    res = jax.lax.dot_general(
        proj_ref[...], e_ref[...], _NT,
        preferred_element_type=jnp.float32) + gb_ref[...]
    out_ref[...] = res.reshape(T, B, res.shape[-1])


def kernel(encoding, trg, E, W_ih0, W_hh0, b_ih0, b_hh0, W_ih1, W_hh1,
           b_ih1, b_hh1, W1, b1, W2, b2, g_b):
    idx = trg.astype(jnp.int32)

    gru_weights = (
        encoding,
        W_ih0, W_hh0, b_ih0.reshape(1, -1), b_hh0.reshape(1, -1),
        W_ih1, W_hh1, b_ih1.reshape(1, -1), b_hh1.reshape(1, -1),
        W1, b1.reshape(1, -1), W2, b2.reshape(1, -1),
    )

    proj = pl.pallas_call(
        _gru_proj_body,
        in_specs=[pl.BlockSpec(memory_space=pltpu.SMEM),
                  pl.BlockSpec(memory_space=pltpu.MemorySpace.HBM)]
                 + [pl.BlockSpec(memory_space=pltpu.MemorySpace.VMEM)] * 13,
        out_shape=jax.ShapeDtypeStruct((BT, EMBED), jnp.float32),
        scratch_shapes=[pltpu.VMEM((BT, EMBED), jnp.float32),
                        pltpu.VMEM((BT, HIDDEN), jnp.float32),
                        pltpu.SemaphoreType.DMA],
    )(idx, E, *gru_weights)

    nv = pl.cdiv(VOCAB, VTILE)
    logits_tb = pl.pallas_call(
        _logits_body,
        grid=(nv,),
        in_specs=[
            pl.BlockSpec((BT, EMBED), lambda i: (0, 0)),
            pl.BlockSpec((VTILE, EMBED), lambda i: (i, 0)),
            pl.BlockSpec((1, VTILE), lambda i: (0, i)),
        ],
        out_specs=pl.BlockSpec((T, B, VTILE), lambda i: (0, 0, i)),
        out_shape=jax.ShapeDtypeStruct((T, B, VOCAB), jnp.float32),
        compiler_params=pltpu.CompilerParams(
            dimension_semantics=("arbitrary",)),
    )(proj, E, g_b.reshape(1, VOCAB))

    return logits_tb.transpose(1, 0, 2)
